# baseline (speedup 1.0000x reference)
import functools

import jax
import jax.numpy as jnp
from jax import lax
from jax.experimental import pallas as pl
from jax.experimental.pallas import tpu as pltpu
from jax.experimental.pallas import tpu_sc as plsc

N = 10000
E = 5000
IN = 128
H = 8
C = 16
HC = H * C
WID = HC + 16
NEG_SLOPE = 0.2

NPAD = 10048
EPAD = 5120
NC = 2
NS = 16
NW = NC * NS
K = 64
NB = 160
NBURST = 2
TOT = NW * NB * K


def _leaky_relu(x):
    return jnp.where(x >= 0, x, NEG_SLOPE * x)


def _s1_body(x_ref, w_ref, o_ref):
    mm = lax.dot_general(x_ref[...], w_ref[...],
                         (((1,), (1,)), ((), ())),
                         preferred_element_type=jnp.float32)
    col = lax.broadcasted_iota(jnp.int32, (mm.shape[0], 16), 1)
    pad = jnp.where(col == 0, 1.0, 0.0).astype(jnp.float32)
    o_ref[...] = jnp.concatenate([mm, pad], axis=1)


def _stage1(xp, w):
    blk = 1256
    grid = NPAD // blk
    return pl.pallas_call(
        _s1_body,
        grid=(grid,),
        in_specs=[
            pl.BlockSpec((blk, IN), lambda i: (i, 0)),
            pl.BlockSpec((HC, IN), lambda i: (0, 0)),
        ],
        out_specs=pl.BlockSpec((blk, WID), lambda i: (i, 0)),
        out_shape=jax.ShapeDtypeStruct((NPAD, WID), jnp.float32),
    )(xp, w)


def _make_sc_pass(rows_acc=NPAD):
    rpt = rows_acc // NS
    mesh = plsc.VectorSubcoreMesh(core_axis_name="c", subcore_axis_name="s")

    @functools.partial(
        pl.kernel,
        out_type=jax.ShapeDtypeStruct((NC, rows_acc, WID), jnp.float32),
        mesh=mesh,
        scratch_types=[
            pltpu.VMEM((NB, K), jnp.int32),
            pltpu.VMEM((NB, K), jnp.int32),
            [pltpu.VMEM((K, WID), jnp.float32)] * NBURST,
            pltpu.VMEM_SHARED((rows_acc, WID), jnp.float32),
            [pltpu.SemaphoreType.DMA] * NBURST,
        ],
        compiler_params=pltpu.CompilerParams(use_tc_tiling_on_sc=False),
    )
    def sc_pass(table_hbm, gidx_hbm, sidx_hbm, zeros_hbm, out_hbm,
                gidx_v, sidx_v, rows_v, acc_sh, sems):
        cid = lax.axis_index("c")
        sid = lax.axis_index("s")
        wid = cid * NS + sid
        pltpu.sync_copy(zeros_hbm.at[pl.ds(0, rpt)],
                        acc_sh.at[pl.ds(sid * rpt, rpt)])
        pltpu.sync_copy(gidx_hbm.at[wid], gidx_v)
        pltpu.sync_copy(sidx_hbm.at[wid], sidx_v)
        plsc.subcore_barrier()

        def body(g, carry):
            bb = g * NBURST
            for r in range(NBURST):
                pltpu.async_copy(table_hbm.at[gidx_v.at[bb + r]],
                                 rows_v[r], sems[r])
            for r in range(NBURST):
                pltpu.make_async_copy(table_hbm.at[gidx_v.at[bb + r]],
                                      rows_v[r], sems[r]).wait()
                pltpu.sync_copy(rows_v[r], acc_sh.at[sidx_v.at[bb + r]],
                                add=True)
            return carry

        lax.fori_loop(0, NB // NBURST, body, 0)
        plsc.subcore_barrier()
        pltpu.sync_copy(acc_sh.at[pl.ds(sid * rpt, rpt)],
                        out_hbm.at[cid, pl.ds(sid * rpt, rpt)])

    return sc_pass


def _s3_body(se_ref, a_ref, b_ref, o_ref):
    s = se_ref[0] + se_ref[1]
    cnt = jnp.maximum(s[:, HC:HC + 1], 1.0)
    xe = s[:, :HC] / cnt
    wbig = jnp.exp(_leaky_relu(jnp.dot(xe, b_ref[...],
                                       preferred_element_type=jnp.float32)))
    w8 = jnp.exp(_leaky_relu(jnp.dot(xe, a_ref[...],
                                     preferred_element_type=jnp.float32)))
    z8 = jnp.zeros((s.shape[0], 8), jnp.float32)
    o_ref[...] = jnp.concatenate([xe * wbig, w8, z8], axis=1)


def _stage3(se, a_mat, b_mat):
    blk = 640
    grid = EPAD // blk
    return pl.pallas_call(
        _s3_body,
        grid=(grid,),
        in_specs=[
            pl.BlockSpec((NC, blk, WID), lambda i: (0, i, 0)),
            pl.BlockSpec((HC, H), lambda i: (0, 0)),
            pl.BlockSpec((HC, HC), lambda i: (0, 0)),
        ],
        out_specs=pl.BlockSpec((blk, WID), lambda i: (i, 0)),
        out_shape=jax.ShapeDtypeStruct((EPAD, WID), jnp.float32),
    )(se, a_mat, b_mat)


def _s5_body(xv_ref, s8_ref, o_ref):
    s = xv_ref[0] + xv_ref[1]
    numer = s[:, :HC]
    den8 = s[:, HC:HC + H]
    dbig = jnp.dot(den8, s8_ref[...], preferred_element_type=jnp.float32)
    out = numer / (dbig + 1e-16)
    nrm = jnp.sqrt(jnp.sum(out * out, axis=1, keepdims=True))
    o_ref[...] = jnp.where(nrm > 0, out / nrm, 0.0)


def _stage5(xv, s8):
    blk = 400
    grid = N // blk
    return pl.pallas_call(
        _s5_body,
        grid=(grid,),
        in_specs=[
            pl.BlockSpec((NC, blk, WID), lambda i: (0, i, 0)),
            pl.BlockSpec((H, HC), lambda i: (0, 0)),
        ],
        out_specs=pl.BlockSpec((blk, HC), lambda i: (i, 0)),
        out_shape=jax.ShapeDtypeStruct((N, HC), jnp.float32),
    )(xv, s8)


def kernel(X, W, att_e, vertex, edges):
    nnz = vertex.shape[0]
    pad = TOT - nnz
    v = jnp.concatenate([vertex.astype(jnp.int32),
                         jnp.full((pad,), NPAD - 1, jnp.int32)])
    e = jnp.concatenate([edges.astype(jnp.int32),
                         jnp.full((pad,), EPAD - 1, jnp.int32)])
    v3 = v.reshape(NW, NB, K)
    e3 = e.reshape(NW, NB, K)

    flat = att_e.reshape(-1).astype(jnp.float32)
    ii = jnp.arange(HC)[:, None]
    b_mat = jnp.where((ii // C) == (jnp.arange(HC)[None, :] // C),
                      flat[:, None], 0.0)
    a_mat = jnp.where((ii // C) == jnp.arange(H)[None, :], flat[:, None], 0.0)
    s8 = jnp.where(jnp.arange(H)[:, None] == (jnp.arange(HC)[None, :] // C),
                   1.0, 0.0).astype(jnp.float32)

    zeros = jnp.zeros((NPAD // NS, WID), jnp.float32)

    xp = jnp.pad(X.astype(jnp.float32), ((0, NPAD - N), (0, 0)))
    x0p = _stage1(xp, W.astype(jnp.float32))
    se = _make_sc_pass(EPAD)(x0p, v3, e3, zeros)
    g = _stage3(se, a_mat, b_mat)
    xv = _make_sc_pass(NPAD)(g, e3, v3, zeros)
    return _stage5(xv, s8)


# speedup vs baseline: 1.0054x; 1.0054x over previous
import functools

import jax
import jax.numpy as jnp
from jax import lax
from jax.experimental import pallas as pl
from jax.experimental.pallas import tpu as pltpu
from jax.experimental.pallas import tpu_sc as plsc

N = 10000
E = 5000
IN = 128
H = 8
C = 16
HC = H * C
WID = HC + 16
NEG_SLOPE = 0.2

NPAD = 10048
EPAD = 5120
NC = 2
NS = 16
NW = NC * NS
K = 128
NB = 80
NBURST = 1
TOT = NW * NB * K


def _leaky_relu(x):
    return jnp.where(x >= 0, x, NEG_SLOPE * x)


def _s1_body(x_ref, w_ref, o_ref):
    mm = lax.dot_general(x_ref[...], w_ref[...],
                         (((1,), (1,)), ((), ())),
                         preferred_element_type=jnp.float32)
    col = lax.broadcasted_iota(jnp.int32, (mm.shape[0], 16), 1)
    pad = jnp.where(col == 0, 1.0, 0.0).astype(jnp.float32)
    o_ref[...] = jnp.concatenate([mm, pad], axis=1)


def _stage1(xp, w):
    blk = 1256
    grid = NPAD // blk
    return pl.pallas_call(
        _s1_body,
        grid=(grid,),
        in_specs=[
            pl.BlockSpec((blk, IN), lambda i: (i, 0)),
            pl.BlockSpec((HC, IN), lambda i: (0, 0)),
        ],
        out_specs=pl.BlockSpec((blk, WID), lambda i: (i, 0)),
        out_shape=jax.ShapeDtypeStruct((NPAD, WID), jnp.float32),
    )(xp, w)


def _make_sc_pass(rows_acc=NPAD):
    rpt = rows_acc // NS
    mesh = plsc.VectorSubcoreMesh(core_axis_name="c", subcore_axis_name="s")

    @functools.partial(
        pl.kernel,
        out_type=jax.ShapeDtypeStruct((NC, rows_acc, WID), jnp.float32),
        mesh=mesh,
        scratch_types=[
            pltpu.VMEM((NB, K), jnp.int32),
            pltpu.VMEM((NB, K), jnp.int32),
            [pltpu.VMEM((K, WID), jnp.float32)] * NBURST,
            pltpu.VMEM_SHARED((rows_acc, WID), jnp.float32),
            [pltpu.SemaphoreType.DMA] * NBURST,
        ],
        compiler_params=pltpu.CompilerParams(use_tc_tiling_on_sc=False),
    )
    def sc_pass(table_hbm, gidx_hbm, sidx_hbm, zeros_hbm, out_hbm,
                gidx_v, sidx_v, rows_v, acc_sh, sems):
        cid = lax.axis_index("c")
        sid = lax.axis_index("s")
        wid = cid * NS + sid
        pltpu.sync_copy(zeros_hbm.at[pl.ds(0, rpt)],
                        acc_sh.at[pl.ds(sid * rpt, rpt)])
        pltpu.sync_copy(gidx_hbm.at[wid], gidx_v)
        pltpu.sync_copy(sidx_hbm.at[wid], sidx_v)
        plsc.subcore_barrier()

        def body(g, carry):
            bb = g * NBURST
            for r in range(NBURST):
                pltpu.async_copy(table_hbm.at[gidx_v.at[bb + r]],
                                 rows_v[r], sems[r])
            for r in range(NBURST):
                pltpu.make_async_copy(table_hbm.at[gidx_v.at[bb + r]],
                                      rows_v[r], sems[r]).wait()
                pltpu.sync_copy(rows_v[r], acc_sh.at[sidx_v.at[bb + r]],
                                add=True)
            return carry

        lax.fori_loop(0, NB // NBURST, body, 0)
        plsc.subcore_barrier()
        pltpu.sync_copy(acc_sh.at[pl.ds(sid * rpt, rpt)],
                        out_hbm.at[cid, pl.ds(sid * rpt, rpt)])

    return sc_pass


def _s3_body(se_ref, a_ref, b_ref, o_ref):
    s = se_ref[0] + se_ref[1]
    cnt = jnp.maximum(s[:, HC:HC + 1], 1.0)
    xe = s[:, :HC] / cnt
    wbig = jnp.exp(_leaky_relu(jnp.dot(xe, b_ref[...],
                                       preferred_element_type=jnp.float32)))
    w8 = jnp.exp(_leaky_relu(jnp.dot(xe, a_ref[...],
                                     preferred_element_type=jnp.float32)))
    z8 = jnp.zeros((s.shape[0], 8), jnp.float32)
    o_ref[...] = jnp.concatenate([xe * wbig, w8, z8], axis=1)


def _stage3(se, a_mat, b_mat):
    blk = 640
    grid = EPAD // blk
    return pl.pallas_call(
        _s3_body,
        grid=(grid,),
        in_specs=[
            pl.BlockSpec((NC, blk, WID), lambda i: (0, i, 0)),
            pl.BlockSpec((HC, H), lambda i: (0, 0)),
            pl.BlockSpec((HC, HC), lambda i: (0, 0)),
        ],
        out_specs=pl.BlockSpec((blk, WID), lambda i: (i, 0)),
        out_shape=jax.ShapeDtypeStruct((EPAD, WID), jnp.float32),
    )(se, a_mat, b_mat)


def _s5_body(xv_ref, s8_ref, o_ref):
    s = xv_ref[0] + xv_ref[1]
    numer = s[:, :HC]
    den8 = s[:, HC:HC + H]
    dbig = jnp.dot(den8, s8_ref[...], preferred_element_type=jnp.float32)
    out = numer / (dbig + 1e-16)
    nrm = jnp.sqrt(jnp.sum(out * out, axis=1, keepdims=True))
    o_ref[...] = jnp.where(nrm > 0, out / nrm, 0.0)


def _stage5(xv, s8):
    blk = 400
    grid = N // blk
    return pl.pallas_call(
        _s5_body,
        grid=(grid,),
        in_specs=[
            pl.BlockSpec((NC, blk, WID), lambda i: (0, i, 0)),
            pl.BlockSpec((H, HC), lambda i: (0, 0)),
        ],
        out_specs=pl.BlockSpec((blk, HC), lambda i: (i, 0)),
        out_shape=jax.ShapeDtypeStruct((N, HC), jnp.float32),
    )(xv, s8)


def kernel(X, W, att_e, vertex, edges):
    nnz = vertex.shape[0]
    pad = TOT - nnz
    v = jnp.concatenate([vertex.astype(jnp.int32),
                         jnp.full((pad,), NPAD - 1, jnp.int32)])
    e = jnp.concatenate([edges.astype(jnp.int32),
                         jnp.full((pad,), EPAD - 1, jnp.int32)])
    v3 = v.reshape(NW, NB, K)
    e3 = e.reshape(NW, NB, K)

    flat = att_e.reshape(-1).astype(jnp.float32)
    ii = jnp.arange(HC)[:, None]
    b_mat = jnp.where((ii // C) == (jnp.arange(HC)[None, :] // C),
                      flat[:, None], 0.0)
    a_mat = jnp.where((ii // C) == jnp.arange(H)[None, :], flat[:, None], 0.0)
    s8 = jnp.where(jnp.arange(H)[:, None] == (jnp.arange(HC)[None, :] // C),
                   1.0, 0.0).astype(jnp.float32)

    zeros = jnp.zeros((NPAD // NS, WID), jnp.float32)

    xp = jnp.pad(X.astype(jnp.float32), ((0, NPAD - N), (0, 0)))
    x0p = _stage1(xp, W.astype(jnp.float32))
    se = _make_sc_pass(EPAD)(x0p, v3, e3, zeros)
    g = _stage3(se, a_mat, b_mat)
    xv = _make_sc_pass(NPAD)(g, e3, v3, zeros)
    return _stage5(xv, s8)


# speedup vs baseline: 1.0153x; 1.0098x over previous
import functools

import jax
import jax.numpy as jnp
from jax import lax
from jax.experimental import pallas as pl
from jax.experimental.pallas import tpu as pltpu
from jax.experimental.pallas import tpu_sc as plsc

N = 10000
E = 5000
IN = 128
H = 8
C = 16
HC = H * C
WID = HC + 16
NEG_SLOPE = 0.2

NPAD = 10048
EPAD = 5120
NC = 2
NS = 16
NW = NC * NS
K = 128
NB = 80
NBURST = 1
TOT = NW * NB * K


def _leaky_relu(x):
    return jnp.where(x >= 0, x, NEG_SLOPE * x)


def _s1_body(x_ref, w_ref, o_ref):
    mm = lax.dot_general(x_ref[...], w_ref[...],
                         (((1,), (1,)), ((), ())),
                         preferred_element_type=jnp.float32)
    col = lax.broadcasted_iota(jnp.int32, (mm.shape[0], 16), 1)
    pad = jnp.where(col == 0, 1.0, 0.0).astype(jnp.float32)
    o_ref[...] = jnp.concatenate([mm, pad], axis=1)


def _stage1(xp, w):
    blk = 1256
    grid = NPAD // blk
    return pl.pallas_call(
        _s1_body,
        grid=(grid,),
        in_specs=[
            pl.BlockSpec((blk, IN), lambda i: (i, 0)),
            pl.BlockSpec((HC, IN), lambda i: (0, 0)),
        ],
        out_specs=pl.BlockSpec((blk, WID), lambda i: (i, 0)),
        out_shape=jax.ShapeDtypeStruct((NPAD, WID), jnp.float32),
    )(xp, w)


def _make_sc_pass(rows_acc=NPAD):
    rpt = rows_acc // NS
    mesh = plsc.VectorSubcoreMesh(core_axis_name="c", subcore_axis_name="s")

    @functools.partial(
        pl.kernel,
        out_type=jax.ShapeDtypeStruct((NC, rows_acc, WID), jnp.float32),
        mesh=mesh,
        scratch_types=[
            [pltpu.VMEM((K,), jnp.int32)] * 2,
            [pltpu.VMEM((K,), jnp.int32)] * 2,
            pltpu.VMEM((K, WID), jnp.float32),
            pltpu.VMEM_SHARED((rows_acc, WID), jnp.float32),
            pltpu.SemaphoreType.DMA,
            [pltpu.SemaphoreType.DMA] * 2,
        ],
        compiler_params=pltpu.CompilerParams(use_tc_tiling_on_sc=False),
    )
    def sc_pass(table_hbm, gidx_hbm, sidx_hbm, zeros_hbm, out_hbm,
                gidx_v, sidx_v, rows_v, acc_sh, sem, isems):
        cid = lax.axis_index("c")
        sid = lax.axis_index("s")
        wid = cid * NS + sid
        pltpu.sync_copy(zeros_hbm.at[pl.ds(0, rpt)],
                        acc_sh.at[pl.ds(sid * rpt, rpt)])
        plsc.subcore_barrier()

        def start_idx(slot, b):
            pltpu.async_copy(gidx_hbm.at[wid, b], gidx_v[slot], isems[slot])
            pltpu.async_copy(sidx_hbm.at[wid, b], sidx_v[slot], isems[slot])

        def wait_idx(slot, b):
            pltpu.make_async_copy(gidx_hbm.at[wid, b], gidx_v[slot],
                                  isems[slot]).wait()
            pltpu.make_async_copy(sidx_hbm.at[wid, b], sidx_v[slot],
                                  isems[slot]).wait()

        def run_batch(slot, b):
            pltpu.async_copy(table_hbm.at[gidx_v[slot]], rows_v, sem).wait()
            pltpu.sync_copy(rows_v, acc_sh.at[sidx_v[slot]], add=True)

        start_idx(0, 0)

        def body(g, carry):
            b0 = 2 * g
            wait_idx(0, b0)
            start_idx(1, b0 + 1)
            run_batch(0, b0)
            wait_idx(1, b0 + 1)

            @pl.when(b0 + 2 < NB)
            def _():
                start_idx(0, b0 + 2)

            run_batch(1, b0 + 1)
            return carry

        lax.fori_loop(0, NB // 2, body, 0)
        plsc.subcore_barrier()
        pltpu.sync_copy(acc_sh.at[pl.ds(sid * rpt, rpt)],
                        out_hbm.at[cid, pl.ds(sid * rpt, rpt)])

    return sc_pass


def _s3_body(se_ref, a_ref, b_ref, o_ref):
    s = se_ref[0] + se_ref[1]
    cnt = jnp.maximum(s[:, HC:HC + 1], 1.0)
    xe = s[:, :HC] / cnt
    wbig = jnp.exp(_leaky_relu(jnp.dot(xe, b_ref[...],
                                       preferred_element_type=jnp.float32)))
    w8 = jnp.exp(_leaky_relu(jnp.dot(xe, a_ref[...],
                                     preferred_element_type=jnp.float32)))
    z8 = jnp.zeros((s.shape[0], 8), jnp.float32)
    o_ref[...] = jnp.concatenate([xe * wbig, w8, z8], axis=1)


def _stage3(se, a_mat, b_mat):
    blk = 640
    grid = EPAD // blk
    return pl.pallas_call(
        _s3_body,
        grid=(grid,),
        in_specs=[
            pl.BlockSpec((NC, blk, WID), lambda i: (0, i, 0)),
            pl.BlockSpec((HC, H), lambda i: (0, 0)),
            pl.BlockSpec((HC, HC), lambda i: (0, 0)),
        ],
        out_specs=pl.BlockSpec((blk, WID), lambda i: (i, 0)),
        out_shape=jax.ShapeDtypeStruct((EPAD, WID), jnp.float32),
    )(se, a_mat, b_mat)


def _s5_body(xv_ref, s8_ref, o_ref):
    s = xv_ref[0] + xv_ref[1]
    numer = s[:, :HC]
    den8 = s[:, HC:HC + H]
    dbig = jnp.dot(den8, s8_ref[...], preferred_element_type=jnp.float32)
    out = numer / (dbig + 1e-16)
    nrm = jnp.sqrt(jnp.sum(out * out, axis=1, keepdims=True))
    o_ref[...] = jnp.where(nrm > 0, out / nrm, 0.0)


def _stage5(xv, s8):
    blk = 400
    grid = N // blk
    return pl.pallas_call(
        _s5_body,
        grid=(grid,),
        in_specs=[
            pl.BlockSpec((NC, blk, WID), lambda i: (0, i, 0)),
            pl.BlockSpec((H, HC), lambda i: (0, 0)),
        ],
        out_specs=pl.BlockSpec((blk, HC), lambda i: (i, 0)),
        out_shape=jax.ShapeDtypeStruct((N, HC), jnp.float32),
    )(xv, s8)


def kernel(X, W, att_e, vertex, edges):
    nnz = vertex.shape[0]
    pad = TOT - nnz
    v = jnp.concatenate([vertex.astype(jnp.int32),
                         jnp.full((pad,), NPAD - 1, jnp.int32)])
    e = jnp.concatenate([edges.astype(jnp.int32),
                         jnp.full((pad,), EPAD - 1, jnp.int32)])
    v3 = v.reshape(NW, NB, K)
    e3 = e.reshape(NW, NB, K)

    flat = att_e.reshape(-1).astype(jnp.float32)
    ii = jnp.arange(HC)[:, None]
    b_mat = jnp.where((ii // C) == (jnp.arange(HC)[None, :] // C),
                      flat[:, None], 0.0)
    a_mat = jnp.where((ii // C) == jnp.arange(H)[None, :], flat[:, None], 0.0)
    s8 = jnp.where(jnp.arange(H)[:, None] == (jnp.arange(HC)[None, :] // C),
                   1.0, 0.0).astype(jnp.float32)

    zeros = jnp.zeros((NPAD // NS, WID), jnp.float32)

    xp = jnp.pad(X.astype(jnp.float32), ((0, NPAD - N), (0, 0)))
    x0p = _stage1(xp, W.astype(jnp.float32))
    se = _make_sc_pass(EPAD)(x0p, v3, e3, zeros)
    g = _stage3(se, a_mat, b_mat)
    xv = _make_sc_pass(NPAD)(g, e3, v3, zeros)
    return _stage5(xv, s8)


# speedup vs baseline: 1.2304x; 1.2118x over previous
import functools

import jax
import jax.numpy as jnp
from jax import lax
from jax.experimental import pallas as pl
from jax.experimental.pallas import tpu as pltpu
from jax.experimental.pallas import tpu_sc as plsc

N = 10000
E = 5000
IN = 128
H = 8
C = 16
HC = H * C
WID = HC + 16
NEG_SLOPE = 0.2

NPAD = 10240
EPAD = 5120
NC = 2
NS = 16
NW = NC * NS
K = 128
NB = 79
TOT = NW * NB * K


def _leaky_relu(x):
    return jnp.where(x >= 0, x, NEG_SLOPE * x)


def _s1_body(x_ref, w_ref, o_ref):
    mm = lax.dot_general(x_ref[...], w_ref[...],
                         (((1,), (1,)), ((), ())),
                         preferred_element_type=jnp.float32)
    col = lax.broadcasted_iota(jnp.int32, (mm.shape[0], 16), 1)
    pad = jnp.where(col == 0, 1.0, 0.0).astype(jnp.float32)
    o_ref[...] = jnp.concatenate([mm, pad], axis=1)


def _stage1(xp, w):
    blk = 1280
    grid = NPAD // blk
    return pl.pallas_call(
        _s1_body,
        grid=(grid,),
        in_specs=[
            pl.BlockSpec((blk, IN), lambda i: (i, 0)),
            pl.BlockSpec((HC, IN), lambda i: (0, 0)),
        ],
        out_specs=pl.BlockSpec((blk, WID), lambda i: (i, 0)),
        out_shape=jax.ShapeDtypeStruct((NPAD, WID), jnp.float32),
    )(xp, w)


def _make_sc_pass(rows_acc=NPAD):
    rpt = rows_acc // NS
    mesh = plsc.VectorSubcoreMesh(core_axis_name="c", subcore_axis_name="s")

    @functools.partial(
        pl.kernel,
        out_type=jax.ShapeDtypeStruct((NC, rows_acc, WID), jnp.float32),
        mesh=mesh,
        scratch_types=[
            pltpu.VMEM((K,), jnp.int32),
            pltpu.VMEM((K,), jnp.int32),
            pltpu.VMEM((K, WID), jnp.float32),
            pltpu.VMEM_SHARED((rows_acc, WID), jnp.float32),
            pltpu.SemaphoreType.DMA,
        ],
        compiler_params=pltpu.CompilerParams(use_tc_tiling_on_sc=False),
    )
    def sc_pass(table_hbm, gidx_hbm, sidx_hbm, zeros_hbm, out_hbm,
                gidx_v, sidx_v, rows_v, acc_sh, sem):
        cid = lax.axis_index("c")
        sid = lax.axis_index("s")
        wid = cid * NS + sid
        pltpu.sync_copy(zeros_hbm.at[pl.ds(0, rpt)],
                        acc_sh.at[pl.ds(sid * rpt, rpt)])
        plsc.subcore_barrier()

        def body(b, carry):
            pltpu.sync_copy(gidx_hbm.at[wid, b], gidx_v)
            pltpu.sync_copy(sidx_hbm.at[wid, b], sidx_v)
            pltpu.async_copy(table_hbm.at[gidx_v], rows_v, sem).wait()
            pltpu.sync_copy(rows_v, acc_sh.at[sidx_v], add=True)
            return carry

        lax.fori_loop(0, NB, body, 0)
        plsc.subcore_barrier()
        pltpu.sync_copy(acc_sh.at[pl.ds(sid * rpt, rpt)],
                        out_hbm.at[cid, pl.ds(sid * rpt, rpt)])

    return sc_pass


def _s3_body(se_ref, a_ref, b_ref, o_ref):
    s = se_ref[0] + se_ref[1]
    cnt = jnp.maximum(s[:, HC:HC + 1], 1.0)
    xe = s[:, :HC] / cnt
    wbig = jnp.exp(_leaky_relu(jnp.dot(xe, b_ref[...],
                                       preferred_element_type=jnp.float32)))
    w8 = jnp.exp(_leaky_relu(jnp.dot(xe, a_ref[...],
                                     preferred_element_type=jnp.float32)))
    z8 = jnp.zeros((s.shape[0], 8), jnp.float32)
    o_ref[...] = jnp.concatenate([xe * wbig, w8, z8], axis=1)


def _stage3(se, a_mat, b_mat):
    blk = 640
    grid = EPAD // blk
    return pl.pallas_call(
        _s3_body,
        grid=(grid,),
        in_specs=[
            pl.BlockSpec((NC, blk, WID), lambda i: (0, i, 0)),
            pl.BlockSpec((HC, H), lambda i: (0, 0)),
            pl.BlockSpec((HC, HC), lambda i: (0, 0)),
        ],
        out_specs=pl.BlockSpec((blk, WID), lambda i: (i, 0)),
        out_shape=jax.ShapeDtypeStruct((EPAD, WID), jnp.float32),
    )(se, a_mat, b_mat)


def _s5_body(xv_ref, s8_ref, o_ref):
    s = xv_ref[0] + xv_ref[1]
    numer = s[:, :HC]
    den8 = s[:, HC:HC + H]
    dbig = jnp.dot(den8, s8_ref[...], preferred_element_type=jnp.float32)
    out = numer / (dbig + 1e-16)
    nrm = jnp.sqrt(jnp.sum(out * out, axis=1, keepdims=True))
    o_ref[...] = jnp.where(nrm > 0, out / nrm, 0.0)


def _stage5(xv, s8):
    blk = 400
    grid = N // blk
    return pl.pallas_call(
        _s5_body,
        grid=(grid,),
        in_specs=[
            pl.BlockSpec((NC, blk, WID), lambda i: (0, i, 0)),
            pl.BlockSpec((H, HC), lambda i: (0, 0)),
        ],
        out_specs=pl.BlockSpec((blk, HC), lambda i: (i, 0)),
        out_shape=jax.ShapeDtypeStruct((N, HC), jnp.float32),
    )(xv, s8)


def kernel(X, W, att_e, vertex, edges):
    nnz = vertex.shape[0]
    pad = TOT - nnz
    v = jnp.concatenate([vertex.astype(jnp.int32),
                         jnp.full((pad,), NPAD - 1, jnp.int32)])
    e = jnp.concatenate([edges.astype(jnp.int32),
                         jnp.full((pad,), EPAD - 1, jnp.int32)])
    v3 = v.reshape(NW, NB, K)
    e3 = e.reshape(NW, NB, K)

    flat = att_e.reshape(-1).astype(jnp.float32)
    ii = jnp.arange(HC)[:, None]
    b_mat = jnp.where((ii // C) == (jnp.arange(HC)[None, :] // C),
                      flat[:, None], 0.0)
    a_mat = jnp.where((ii // C) == jnp.arange(H)[None, :], flat[:, None], 0.0)
    s8 = jnp.where(jnp.arange(H)[:, None] == (jnp.arange(HC)[None, :] // C),
                   1.0, 0.0).astype(jnp.float32)

    zeros = jnp.zeros((NPAD // NS, WID), jnp.float32)

    xp = jnp.pad(X.astype(jnp.float32), ((0, NPAD - N), (0, 0)))
    x0p = _stage1(xp, W.astype(jnp.float32))
    se = _make_sc_pass(EPAD)(x0p, v3, e3, zeros)
    g = _stage3(se, a_mat, b_mat)
    xv = _make_sc_pass(NPAD)(g, e3, v3, zeros)
    return _stage5(xv, s8)


# speedup vs baseline: 1.2933x; 1.0511x over previous
import functools

import jax
import jax.numpy as jnp
from jax import lax
from jax.experimental import pallas as pl
from jax.experimental.pallas import tpu as pltpu
from jax.experimental.pallas import tpu_sc as plsc

N = 10000
E = 5000
IN = 128
H = 8
C = 16
HC = H * C
WID = HC + 16
NEG_SLOPE = 0.2

NPAD = 10048
EPAD = 5120
NC = 2
NS = 16
NW = NC * NS
K = 128
NB = 79
TOT = NW * NB * K


def _leaky_relu(x):
    return jnp.where(x >= 0, x, NEG_SLOPE * x)


def _s1_body(x_ref, w_ref, o_ref):
    mm = lax.dot_general(x_ref[...], w_ref[...],
                         (((1,), (1,)), ((), ())),
                         preferred_element_type=jnp.float32)
    col = lax.broadcasted_iota(jnp.int32, (mm.shape[0], 16), 1)
    pad = jnp.where(col == 0, 1.0, 0.0).astype(jnp.float32)
    o_ref[...] = jnp.concatenate([mm, pad], axis=1)


def _stage1(xp, w):
    blk = 1256
    grid = NPAD // blk
    return pl.pallas_call(
        _s1_body,
        grid=(grid,),
        in_specs=[
            pl.BlockSpec((blk, IN), lambda i: (i, 0)),
            pl.BlockSpec((HC, IN), lambda i: (0, 0)),
        ],
        out_specs=pl.BlockSpec((blk, WID), lambda i: (i, 0)),
        out_shape=jax.ShapeDtypeStruct((NPAD, WID), jnp.float32),
    )(xp, w)


def _make_sc_pass(rows_acc=NPAD):
    rpt = rows_acc // NS
    mesh = plsc.VectorSubcoreMesh(core_axis_name="c", subcore_axis_name="s")

    @functools.partial(
        pl.kernel,
        out_type=jax.ShapeDtypeStruct((NC, rows_acc, WID), jnp.float32),
        mesh=mesh,
        scratch_types=[
            pltpu.VMEM((K,), jnp.int32),
            pltpu.VMEM((K,), jnp.int32),
            pltpu.VMEM((K, WID), jnp.float32),
            pltpu.VMEM_SHARED((rows_acc, WID), jnp.float32),
            pltpu.SemaphoreType.DMA,
        ],
        compiler_params=pltpu.CompilerParams(use_tc_tiling_on_sc=False),
    )
    def sc_pass(table_hbm, gidx_hbm, sidx_hbm, zeros_hbm, out_hbm,
                gidx_v, sidx_v, rows_v, acc_sh, sem):
        cid = lax.axis_index("c")
        sid = lax.axis_index("s")
        wid = cid * NS + sid
        pltpu.sync_copy(zeros_hbm.at[pl.ds(0, rpt)],
                        acc_sh.at[pl.ds(sid * rpt, rpt)])
        plsc.subcore_barrier()

        def body(b, carry):
            pltpu.sync_copy(gidx_hbm.at[wid, b], gidx_v)
            pltpu.sync_copy(sidx_hbm.at[wid, b], sidx_v)
            pltpu.async_copy(table_hbm.at[gidx_v], rows_v, sem).wait()
            pltpu.sync_copy(rows_v, acc_sh.at[sidx_v], add=True)
            return carry

        lax.fori_loop(0, NB, body, 0)
        plsc.subcore_barrier()
        pltpu.sync_copy(acc_sh.at[pl.ds(sid * rpt, rpt)],
                        out_hbm.at[cid, pl.ds(sid * rpt, rpt)])

    return sc_pass


def _s3_body(se_ref, a_ref, b_ref, o_ref):
    s = se_ref[0] + se_ref[1]
    cnt = jnp.maximum(s[:, HC:HC + 1], 1.0)
    xe = s[:, :HC] / cnt
    wbig = jnp.exp(_leaky_relu(jnp.dot(xe, b_ref[...],
                                       preferred_element_type=jnp.float32)))
    w8 = jnp.exp(_leaky_relu(jnp.dot(xe, a_ref[...],
                                     preferred_element_type=jnp.float32)))
    z8 = jnp.zeros((s.shape[0], 8), jnp.float32)
    o_ref[...] = jnp.concatenate([xe * wbig, w8, z8], axis=1)


def _stage3(se, a_mat, b_mat):
    blk = 640
    grid = EPAD // blk
    return pl.pallas_call(
        _s3_body,
        grid=(grid,),
        in_specs=[
            pl.BlockSpec((NC, blk, WID), lambda i: (0, i, 0)),
            pl.BlockSpec((HC, H), lambda i: (0, 0)),
            pl.BlockSpec((HC, HC), lambda i: (0, 0)),
        ],
        out_specs=pl.BlockSpec((blk, WID), lambda i: (i, 0)),
        out_shape=jax.ShapeDtypeStruct((EPAD, WID), jnp.float32),
    )(se, a_mat, b_mat)


def _s5_body(xv_ref, s8_ref, o_ref):
    s = xv_ref[0] + xv_ref[1]
    numer = s[:, :HC]
    den8 = s[:, HC:HC + H]
    dbig = jnp.dot(den8, s8_ref[...], preferred_element_type=jnp.float32)
    out = numer / (dbig + 1e-16)
    nrm = jnp.sqrt(jnp.sum(out * out, axis=1, keepdims=True))
    o_ref[...] = jnp.where(nrm > 0, out / nrm, 0.0)


def _stage5(xv, s8):
    blk = 400
    grid = N // blk
    return pl.pallas_call(
        _s5_body,
        grid=(grid,),
        in_specs=[
            pl.BlockSpec((NC, blk, WID), lambda i: (0, i, 0)),
            pl.BlockSpec((H, HC), lambda i: (0, 0)),
        ],
        out_specs=pl.BlockSpec((blk, HC), lambda i: (i, 0)),
        out_shape=jax.ShapeDtypeStruct((N, HC), jnp.float32),
    )(xv, s8)


def kernel(X, W, att_e, vertex, edges):
    nnz = vertex.shape[0]
    pad = TOT - nnz
    v = jnp.concatenate([vertex.astype(jnp.int32),
                         jnp.full((pad,), NPAD - 1, jnp.int32)])
    e = jnp.concatenate([edges.astype(jnp.int32),
                         jnp.full((pad,), EPAD - 1, jnp.int32)])
    v3 = v.reshape(NW, NB, K)
    e3 = e.reshape(NW, NB, K)

    flat = att_e.reshape(-1).astype(jnp.float32)
    ii = jnp.arange(HC)[:, None]
    b_mat = jnp.where((ii // C) == (jnp.arange(HC)[None, :] // C),
                      flat[:, None], 0.0)
    a_mat = jnp.where((ii // C) == jnp.arange(H)[None, :], flat[:, None], 0.0)
    s8 = jnp.where(jnp.arange(H)[:, None] == (jnp.arange(HC)[None, :] // C),
                   1.0, 0.0).astype(jnp.float32)

    zeros = jnp.zeros((NPAD // NS, WID), jnp.float32)

    xp = jnp.pad(X.astype(jnp.float32), ((0, NPAD - N), (0, 0)))
    x0p = _stage1(xp, W.astype(jnp.float32))
    se = _make_sc_pass(EPAD)(x0p, v3, e3, zeros)
    g = _stage3(se, a_mat, b_mat)
    xv = _make_sc_pass(NPAD)(g, e3, v3, zeros)
    return _stage5(xv, s8)


# speedup vs baseline: 1.4191x; 1.0973x over previous
import functools

import jax
import jax.numpy as jnp
from jax import lax
from jax.experimental import pallas as pl
from jax.experimental.pallas import tpu as pltpu
from jax.experimental.pallas import tpu_sc as plsc

N = 10000
E = 5000
IN = 128
H = 8
C = 16
HC = H * C
WID = HC + 16
NEG_SLOPE = 0.2

NPAD = 10048
EPAD = 5120
NC = 2
NS = 16
NW = NC * NS
K = 128
NB = 79
TOT = NW * NB * K


def _leaky_relu(x):
    return jnp.where(x >= 0, x, NEG_SLOPE * x)


def _s1_body(x_ref, w_ref, o_ref):
    mm = lax.dot_general(x_ref[...], w_ref[...],
                         (((1,), (1,)), ((), ())),
                         preferred_element_type=jnp.float32)
    col = lax.broadcasted_iota(jnp.int32, (mm.shape[0], 16), 1)
    pad = jnp.where(col == 0, 1.0, 0.0).astype(jnp.float32)
    o_ref[...] = jnp.concatenate([mm, pad], axis=1)


def _stage1(xp, w):
    blk = 1256
    grid = NPAD // blk
    return pl.pallas_call(
        _s1_body,
        grid=(grid,),
        in_specs=[
            pl.BlockSpec((blk, IN), lambda i: (i, 0)),
            pl.BlockSpec((HC, IN), lambda i: (0, 0)),
        ],
        out_specs=pl.BlockSpec((blk, WID), lambda i: (i, 0)),
        out_shape=jax.ShapeDtypeStruct((NPAD, WID), jnp.float32),
    )(xp, w)


def _make_sc_pass(rows_acc=NPAD):
    rpt = rows_acc // NS
    mesh = plsc.VectorSubcoreMesh(core_axis_name="c", subcore_axis_name="s")

    @functools.partial(
        pl.kernel,
        out_type=jax.ShapeDtypeStruct((NC, rows_acc, WID), jnp.float32),
        mesh=mesh,
        scratch_types=[
            pltpu.VMEM((2, K), jnp.int32),
            pltpu.VMEM((K, WID), jnp.float32),
            pltpu.VMEM_SHARED((rows_acc, WID), jnp.float32),
            pltpu.SemaphoreType.DMA,
        ],
        compiler_params=pltpu.CompilerParams(use_tc_tiling_on_sc=False),
    )
    def sc_pass(table_hbm, midx_hbm, zeros_hbm, out_hbm,
                idx_v, rows_v, acc_sh, sem):
        cid = lax.axis_index("c")
        sid = lax.axis_index("s")
        wid = cid * NS + sid
        pltpu.sync_copy(zeros_hbm.at[pl.ds(0, rpt)],
                        acc_sh.at[pl.ds(sid * rpt, rpt)])
        plsc.subcore_barrier()

        def body(b, carry):
            pltpu.sync_copy(midx_hbm.at[wid, b], idx_v)
            pltpu.async_copy(table_hbm.at[idx_v.at[0]], rows_v, sem).wait()
            pltpu.sync_copy(rows_v, acc_sh.at[idx_v.at[1]], add=True)
            return carry

        lax.fori_loop(0, NB, body, 0)
        plsc.subcore_barrier()
        pltpu.sync_copy(acc_sh.at[pl.ds(sid * rpt, rpt)],
                        out_hbm.at[cid, pl.ds(sid * rpt, rpt)])

    return sc_pass


def _s3_body(se_ref, a_ref, b_ref, o_ref):
    s = se_ref[0] + se_ref[1]
    cnt = jnp.maximum(s[:, HC:HC + 1], 1.0)
    xe = s[:, :HC] / cnt
    wbig = jnp.exp(_leaky_relu(jnp.dot(xe, b_ref[...],
                                       preferred_element_type=jnp.float32)))
    w8 = jnp.exp(_leaky_relu(jnp.dot(xe, a_ref[...],
                                     preferred_element_type=jnp.float32)))
    z8 = jnp.zeros((s.shape[0], 8), jnp.float32)
    o_ref[...] = jnp.concatenate([xe * wbig, w8, z8], axis=1)


def _stage3(se, a_mat, b_mat):
    blk = 640
    grid = EPAD // blk
    return pl.pallas_call(
        _s3_body,
        grid=(grid,),
        in_specs=[
            pl.BlockSpec((NC, blk, WID), lambda i: (0, i, 0)),
            pl.BlockSpec((HC, H), lambda i: (0, 0)),
            pl.BlockSpec((HC, HC), lambda i: (0, 0)),
        ],
        out_specs=pl.BlockSpec((blk, WID), lambda i: (i, 0)),
        out_shape=jax.ShapeDtypeStruct((EPAD, WID), jnp.float32),
    )(se, a_mat, b_mat)


def _s5_body(xv_ref, s8_ref, o_ref):
    s = xv_ref[0] + xv_ref[1]
    numer = s[:, :HC]
    den8 = s[:, HC:HC + H]
    dbig = jnp.dot(den8, s8_ref[...], preferred_element_type=jnp.float32)
    out = numer / (dbig + 1e-16)
    nrm = jnp.sqrt(jnp.sum(out * out, axis=1, keepdims=True))
    o_ref[...] = jnp.where(nrm > 0, out / nrm, 0.0)


def _stage5(xv, s8):
    blk = 400
    grid = N // blk
    return pl.pallas_call(
        _s5_body,
        grid=(grid,),
        in_specs=[
            pl.BlockSpec((NC, blk, WID), lambda i: (0, i, 0)),
            pl.BlockSpec((H, HC), lambda i: (0, 0)),
        ],
        out_specs=pl.BlockSpec((blk, HC), lambda i: (i, 0)),
        out_shape=jax.ShapeDtypeStruct((N, HC), jnp.float32),
    )(xv, s8)


def kernel(X, W, att_e, vertex, edges):
    nnz = vertex.shape[0]
    pad = TOT - nnz
    v = jnp.concatenate([vertex.astype(jnp.int32),
                         jnp.full((pad,), NPAD - 1, jnp.int32)])
    e = jnp.concatenate([edges.astype(jnp.int32),
                         jnp.full((pad,), EPAD - 1, jnp.int32)])
    v3 = v.reshape(NW, NB, K)
    e3 = e.reshape(NW, NB, K)
    m_ve = jnp.stack([v3, e3], axis=2)
    m_ev = jnp.stack([e3, v3], axis=2)

    flat = att_e.reshape(-1).astype(jnp.float32)
    ii = jnp.arange(HC)[:, None]
    b_mat = jnp.where((ii // C) == (jnp.arange(HC)[None, :] // C),
                      flat[:, None], 0.0)
    a_mat = jnp.where((ii // C) == jnp.arange(H)[None, :], flat[:, None], 0.0)
    s8 = jnp.where(jnp.arange(H)[:, None] == (jnp.arange(HC)[None, :] // C),
                   1.0, 0.0).astype(jnp.float32)

    zeros = jnp.zeros((NPAD // NS, WID), jnp.float32)

    xp = jnp.pad(X.astype(jnp.float32), ((0, NPAD - N), (0, 0)))
    x0p = _stage1(xp, W.astype(jnp.float32))
    se = _make_sc_pass(EPAD)(x0p, m_ve, zeros)
    g = _stage3(se, a_mat, b_mat)
    xv = _make_sc_pass(NPAD)(g, m_ev, zeros)
    return _stage5(xv, s8)
